# baseline (speedup 1.0000x reference)
import functools

import jax
import jax.numpy as jnp
from jax.experimental import pallas as pl
from jax.experimental.pallas import tpu as pltpu

_K = 16
_MAX_RADIUS = 1.5


def _knn_block(q_ref, q2_ref, k2_ref, kt_ref, dist_ref, idx_ref, *, m_total):
    q = q_ref[...]
    q2 = q2_ref[:, 0:1]
    k2 = k2_ref[0:1, :]
    kt = kt_ref[...]
    qk = jax.lax.dot_general(q, kt, (((1,), (1,)), ((), ())),
                             preferred_element_type=jnp.float32)
    d2 = jnp.maximum(q2 + k2 - 2.0 * qk, 0.0)
    colf = jax.lax.broadcasted_iota(jnp.int32, d2.shape, 1).astype(jnp.float32)
    big = jnp.float32(m_total)
    work = d2
    vals = []
    idxs = []
    for j in range(_K):
        m = jnp.min(work, axis=1, keepdims=True)
        a = jnp.min(jnp.where(work == m, colf, big), axis=1, keepdims=True)
        if j + 1 < _K:
            work = jnp.where(colf == a, jnp.inf, work)
        vals.append(m)
        idxs.append(a)
    v = jnp.concatenate(vals, axis=1)
    i = jnp.concatenate(idxs, axis=1).astype(jnp.int32)
    dist = jnp.sqrt(v)
    masked = jnp.where(dist > _MAX_RADIUS, jnp.broadcast_to(i[:, :1], i.shape), i)
    dist_ref[...] = dist
    idx_ref[...] = masked


def _knn(pc1, pred_flow, pc2, bn=256, interpret=False):
    n = pc1.shape[1]
    m = pc2.shape[1]
    deformed = pc1 + pred_flow
    q = deformed[0]
    kpts = pc2[0]
    q2 = jnp.sum(q * q, axis=-1, keepdims=True)
    k2 = jnp.sum(kpts * kpts, axis=-1)[None, :]
    q2b = jnp.broadcast_to(q2, (n, 8))
    k2b = jnp.broadcast_to(k2, (8, m))
    return pl.pallas_call(
        functools.partial(_knn_block, m_total=m),
        grid=(n // bn,),
        in_specs=[
            pl.BlockSpec((bn, 3), lambda i: (i, 0)),
            pl.BlockSpec((bn, 8), lambda i: (i, 0)),
            pl.BlockSpec((8, m), lambda i: (0, 0)),
            pl.BlockSpec((m, 3), lambda i: (0, 0)),
        ],
        out_specs=[
            pl.BlockSpec((bn, _K), lambda i: (i, 0)),
            pl.BlockSpec((bn, _K), lambda i: (i, 0)),
        ],
        out_shape=[
            jax.ShapeDtypeStruct((n, _K), jnp.float32),
            jax.ShapeDtypeStruct((n, _K), jnp.int32),
        ],
        compiler_params=pltpu.CompilerParams(
            dimension_semantics=("parallel",),
        ),
        interpret=interpret,
    )(q, q2b, k2b, kpts)


def kernel(pc1, pred_flow, pc2):
    dist, idx = _knn(pc1, pred_flow, pc2)
    return dist, idx


# speedup vs baseline: 1.0115x; 1.0115x over previous
import functools

import jax
import jax.numpy as jnp
from jax.experimental import pallas as pl
from jax.experimental.pallas import tpu as pltpu

_K = 16
_MAX_RADIUS = 1.5


def _knn_block(q_ref, q2_ref, k2_ref, kt_ref, dist_ref, idx_ref, *, m_total):
    q = q_ref[...]
    q2 = q2_ref[:, 0:1]
    k2 = k2_ref[0:1, :]
    kt = kt_ref[...]
    qk = jnp.dot(q, kt, preferred_element_type=jnp.float32)
    d2 = jnp.maximum(q2 + k2 - 2.0 * qk, 0.0)
    colf = jax.lax.broadcasted_iota(jnp.int32, d2.shape, 1).astype(jnp.float32)
    big = jnp.float32(m_total)
    work = d2
    vals = []
    idxs = []
    for j in range(_K):
        m = jnp.min(work, axis=1, keepdims=True)
        a = jnp.min(jnp.where(work == m, colf, big), axis=1, keepdims=True)
        if j + 1 < _K:
            work = jnp.where(colf == a, jnp.inf, work)
        vals.append(m)
        idxs.append(a)
    v = jnp.concatenate(vals, axis=1)
    i = jnp.concatenate(idxs, axis=1).astype(jnp.int32)
    dist = jnp.sqrt(v)
    masked = jnp.where(dist > _MAX_RADIUS, jnp.broadcast_to(i[:, :1], i.shape), i)
    dist_ref[...] = dist
    idx_ref[...] = masked


def _knn(pc1, pred_flow, pc2, bn=256, interpret=False):
    n = pc1.shape[1]
    m = pc2.shape[1]
    deformed = pc1 + pred_flow
    q = deformed[0]
    kpts = pc2[0]
    q2 = jnp.sum(q * q, axis=-1, keepdims=True)
    k2 = jnp.sum(kpts * kpts, axis=-1)[None, :]
    q2b = jnp.broadcast_to(q2, (n, 8))
    k2b = jnp.broadcast_to(k2, (8, m))
    kt = kpts.T
    return pl.pallas_call(
        functools.partial(_knn_block, m_total=m),
        grid=(n // bn,),
        in_specs=[
            pl.BlockSpec((bn, 3), lambda i: (i, 0)),
            pl.BlockSpec((bn, 8), lambda i: (i, 0)),
            pl.BlockSpec((8, m), lambda i: (0, 0)),
            pl.BlockSpec((3, m), lambda i: (0, 0)),
        ],
        out_specs=[
            pl.BlockSpec((bn, _K), lambda i: (i, 0)),
            pl.BlockSpec((bn, _K), lambda i: (i, 0)),
        ],
        out_shape=[
            jax.ShapeDtypeStruct((n, _K), jnp.float32),
            jax.ShapeDtypeStruct((n, _K), jnp.int32),
        ],
        compiler_params=pltpu.CompilerParams(
            dimension_semantics=("parallel",),
        ),
        interpret=interpret,
    )(q, q2b, k2b, kt)


def kernel(pc1, pred_flow, pc2):
    dist, idx = _knn(pc1, pred_flow, pc2)
    return dist, idx
